# baseline (speedup 1.0000x reference)
import functools

import jax
import jax.numpy as jnp
import numpy as np
from jax.experimental import pallas as pl
from jax.experimental.pallas import tpu as pltpu


def _upsample_matrix(n):
    o = np.arange(2 * n)
    src = np.clip((o.astype(np.float32) + 0.5) * 0.5 - 0.5, 0.0, float(n - 1))
    i0 = np.floor(src).astype(np.int32)
    i1 = np.minimum(i0 + 1, n - 1)
    lam = (src - i0.astype(np.float32)).astype(np.float32)
    u = np.zeros((n, 2 * n), np.float32)
    np.add.at(u, (i0, o), 1.0 - lam)
    np.add.at(u, (i1, o), lam)
    return jnp.asarray(u)


def _y_stats_kernel(w_ref, x_ref, skip_ref, uw_ref, uht_ref,
                    y_ref, sum_ref, ssq_ref, *, n_ci, n_co, d, h, wd):
    n = pl.program_id(0)
    h2, w2 = 2 * h, 2 * wd
    lanes = n_co * d * w2

    @pl.when(n == 0)
    def _init():
        sum_ref[...] = jnp.zeros_like(sum_ref)
        ssq_ref[...] = jnp.zeros_like(ssq_ref)

    xs = [x_ref[0, ci].reshape(d * h, wd).astype(jnp.bfloat16)
          for ci in range(n_ci)]
    z_list = []
    for c in range(n_co):
        z = xs[0] * w_ref[c, 0].astype(jnp.bfloat16)
        for ci in range(1, n_ci):
            z = z + xs[ci] * w_ref[c, ci].astype(jnp.bfloat16)
        z_list.append(z)
    zcat = jnp.concatenate(z_list, axis=0)

    t = jnp.dot(zcat, uw_ref[...],
                preferred_element_type=jnp.float32)
    t2 = jnp.concatenate(
        [t[i * h:(i + 1) * h] for i in range(n_co * d)], axis=1)

    y = jnp.dot(uht_ref[...], t2,
                preferred_element_type=jnp.float32)

    skipcat = jnp.concatenate(
        [skip_ref[0, c, dd] for c in range(n_co) for dd in range(d)],
        axis=1)
    y = y + skipcat

    yr = y.reshape(h2 // 8, 8, lanes)
    sum_ref[...] += jnp.sum(yr, axis=0)
    ssq_ref[...] += jnp.sum(yr * yr, axis=0)

    for c in range(n_co):
        for dd in range(d):
            i = c * d + dd
            y_ref[0, c, dd] = y[:, i * w2:(i + 1) * w2]


def _bn_elu_kernel(scale_ref, shift_ref, y_ref, out_ref, *, n_co, pair, d):
    for j in range(pair):
        for c in range(n_co):
            for dd in range(d):
                t = y_ref[j, c, dd] * scale_ref[c] + shift_ref[c]
                out_ref[j, c, dd] = jnp.where(
                    t > 0, t, jnp.exp(jnp.minimum(t, 0.0)) - 1.0)


def kernel(x, skip, w, b, *, eps=1e-5):
    n_n, n_ci, d, h, wd = x.shape
    n_co = w.shape[0]
    h2, w2 = 2 * h, 2 * wd
    lanes = n_co * d * w2
    pair = 2
    del b

    x = x.astype(jnp.float32)
    skip = skip.astype(jnp.float32)
    w32 = w.astype(jnp.float32)

    uw = _upsample_matrix(wd).astype(jnp.bfloat16)
    uht = _upsample_matrix(h).T

    smem_spec = pl.BlockSpec(memory_space=pltpu.MemorySpace.SMEM)
    x_spec = pl.BlockSpec((1, n_ci, d, h, wd), lambda n: (n, 0, 0, 0, 0))
    skip_spec = pl.BlockSpec((1, n_co, d, h2, w2), lambda n: (n, 0, 0, 0, 0))
    y_spec = pl.BlockSpec((1, n_co, d, h2, w2), lambda n: (n, 0, 0, 0, 0))
    stat_spec = pl.BlockSpec((8, lanes), lambda n: (0, 0))
    uw_spec = pl.BlockSpec((wd, w2), lambda n: (0, 0))
    uht_spec = pl.BlockSpec((h2, h), lambda n: (0, 0))

    y, csum, cssq = pl.pallas_call(
        functools.partial(_y_stats_kernel, n_ci=n_ci, n_co=n_co,
                          d=d, h=h, wd=wd),
        out_shape=(jax.ShapeDtypeStruct((n_n, n_co, d, h2, w2), jnp.float32),
                   jax.ShapeDtypeStruct((8, lanes), jnp.float32),
                   jax.ShapeDtypeStruct((8, lanes), jnp.float32)),
        grid=(n_n,),
        in_specs=[smem_spec, x_spec, skip_spec, uw_spec, uht_spec],
        out_specs=(y_spec, stat_spec, stat_spec),
        compiler_params=pltpu.CompilerParams(
            dimension_semantics=("arbitrary",)),
    )(w32, x, skip, uw, uht)

    cnt = jnp.float32(n_n * d * h2 * w2)
    per_c = csum.reshape(8, n_co, d * w2)
    per_q = cssq.reshape(8, n_co, d * w2)
    mean = jnp.sum(per_c, axis=(0, 2)) / cnt
    var = jnp.maximum(jnp.sum(per_q, axis=(0, 2)) / cnt - mean * mean, 0.0)
    scale = 1.0 / jnp.sqrt(var + eps)
    shift = -mean * scale

    pair_spec = pl.BlockSpec((pair, n_co, d, h2, w2),
                             lambda g: (g, 0, 0, 0, 0))
    out = pl.pallas_call(
        functools.partial(_bn_elu_kernel, n_co=n_co, pair=pair, d=d),
        out_shape=jax.ShapeDtypeStruct((n_n, n_co, d, h2, w2), jnp.float32),
        grid=(n_n // pair,),
        in_specs=[smem_spec, smem_spec, pair_spec],
        out_specs=pair_spec,
        compiler_params=pltpu.CompilerParams(
            dimension_semantics=("arbitrary",)),
    )(scale, shift, y)
    return out


# speedup vs baseline: 1.6041x; 1.6041x over previous
import functools

import jax
import jax.numpy as jnp
import numpy as np
from jax.experimental import pallas as pl
from jax.experimental.pallas import tpu as pltpu


def _upsample_matrix(n):
    o = np.arange(2 * n)
    src = np.clip((o.astype(np.float32) + 0.5) * 0.5 - 0.5, 0.0, float(n - 1))
    i0 = np.floor(src).astype(np.int32)
    i1 = np.minimum(i0 + 1, n - 1)
    lam = (src - i0.astype(np.float32)).astype(np.float32)
    u = np.zeros((n, 2 * n), np.float32)
    np.add.at(u, (i0, o), 1.0 - lam)
    np.add.at(u, (i1, o), lam)
    return jnp.asarray(u)


def _fused_kernel(w_ref, x_ref, skip_ref, uw_ref, uht_ref, out_ref,
                  y_sc, sum_sc, ssq_sc, scale_sc, shift_sc,
                  *, n_ci, n_co, n_n, d, h, wd, eps):
    p = pl.program_id(0)
    n = pl.program_id(1)
    h2, w2 = 2 * h, 2 * wd
    lanes = n_co * d * w2

    @pl.when((p == 0) & (n == 0))
    def _init():
        sum_sc[...] = jnp.zeros_like(sum_sc)
        ssq_sc[...] = jnp.zeros_like(ssq_sc)

    @pl.when(p == 0)
    def _compute():
        xs = [x_ref[0, ci].reshape(d * h, wd).astype(jnp.bfloat16)
              for ci in range(n_ci)]
        z_list = []
        for c in range(n_co):
            z = xs[0] * w_ref[c, 0].astype(jnp.bfloat16)
            for ci in range(1, n_ci):
                z = z + xs[ci] * w_ref[c, ci].astype(jnp.bfloat16)
            z_list.append(z)
        zcat = jnp.concatenate(z_list, axis=0)

        t = jnp.dot(zcat, uw_ref[...],
                    preferred_element_type=jnp.float32)
        t2 = jnp.concatenate(
            [t[i * h:(i + 1) * h] for i in range(n_co * d)], axis=1)

        y = jnp.dot(uht_ref[...], t2,
                    preferred_element_type=jnp.float32)

        skipcat = jnp.concatenate(
            [skip_ref[0, c, dd] for c in range(n_co) for dd in range(d)],
            axis=1)
        y = y + skipcat

        y_sc[n] = y
        yr = y.reshape(h2 // 8, 8, lanes)
        sum_sc[...] += jnp.sum(yr, axis=0)
        ssq_sc[...] += jnp.sum(yr * yr, axis=0)

    @pl.when((p == 1) & (n == 0))
    def _finalize_stats():
        cnt = jnp.float32(n_n * d * h2 * w2)
        sc_parts, sh_parts = [], []
        for c in range(n_co):
            sl = slice(c * d * w2, (c + 1) * d * w2)
            s = jnp.sum(sum_sc[:, sl])
            q = jnp.sum(ssq_sc[:, sl])
            mean = s / cnt
            var = jnp.maximum(q / cnt - mean * mean, 0.0)
            scl = jax.lax.rsqrt(var + eps)
            sc_parts.append(jnp.full((8, d * w2), scl, jnp.float32))
            sh_parts.append(jnp.full((8, d * w2), -mean * scl, jnp.float32))
        scale_sc[...] = jnp.concatenate(sc_parts, axis=1)
        shift_sc[...] = jnp.concatenate(sh_parts, axis=1)

    @pl.when(p == 1)
    def _apply():
        y = y_sc[n].reshape(h2 // 8, 8, lanes)
        t = (y * scale_sc[...] + shift_sc[...]).reshape(h2, lanes)
        r = jnp.where(t > 0, t, jnp.exp(jnp.minimum(t, 0.0)) - 1.0)
        for c in range(n_co):
            for dd in range(d):
                i = c * d + dd
                out_ref[0, c, dd] = r[:, i * w2:(i + 1) * w2]


def kernel(x, skip, w, b, *, eps=1e-5):
    n_n, n_ci, d, h, wd = x.shape
    n_co = w.shape[0]
    h2, w2 = 2 * h, 2 * wd
    lanes = n_co * d * w2
    del b

    x = x.astype(jnp.float32)
    skip = skip.astype(jnp.float32)
    w32 = w.astype(jnp.float32)

    uw = _upsample_matrix(wd).astype(jnp.bfloat16)
    uht = _upsample_matrix(h).T

    grid = (2, n_n)

    smem_spec = pl.BlockSpec(memory_space=pltpu.MemorySpace.SMEM)
    x_spec = pl.BlockSpec((1, n_ci, d, h, wd),
                          lambda p, n: ((1 - p) * n, 0, 0, 0, 0))
    skip_spec = pl.BlockSpec((1, n_co, d, h2, w2),
                             lambda p, n: ((1 - p) * n, 0, 0, 0, 0))
    out_spec = pl.BlockSpec((1, n_co, d, h2, w2),
                            lambda p, n: (p * n, 0, 0, 0, 0))
    uw_spec = pl.BlockSpec((wd, w2), lambda p, n: (0, 0))
    uht_spec = pl.BlockSpec((h2, h), lambda p, n: (0, 0))

    return pl.pallas_call(
        functools.partial(_fused_kernel, n_ci=n_ci, n_co=n_co,
                          n_n=n_n, d=d, h=h, wd=wd, eps=eps),
        out_shape=jax.ShapeDtypeStruct((n_n, n_co, d, h2, w2), jnp.float32),
        grid=grid,
        in_specs=[smem_spec, x_spec, skip_spec, uw_spec, uht_spec],
        out_specs=out_spec,
        scratch_shapes=[
            pltpu.VMEM((n_n, h2, lanes), jnp.float32),
            pltpu.VMEM((8, lanes), jnp.float32),
            pltpu.VMEM((8, lanes), jnp.float32),
            pltpu.VMEM((8, lanes), jnp.float32),
            pltpu.VMEM((8, lanes), jnp.float32),
        ],
        compiler_params=pltpu.CompilerParams(
            dimension_semantics=("arbitrary", "arbitrary")),
    )(w32, x, skip, uw, uht)


# speedup vs baseline: 1.6626x; 1.0365x over previous
import functools

import jax
import jax.numpy as jnp
import numpy as np
from jax.experimental import pallas as pl
from jax.experimental.pallas import tpu as pltpu


def _upsample_matrix(n):
    o = np.arange(2 * n)
    src = np.clip((o.astype(np.float32) + 0.5) * 0.5 - 0.5, 0.0, float(n - 1))
    i0 = np.floor(src).astype(np.int32)
    i1 = np.minimum(i0 + 1, n - 1)
    lam = (src - i0.astype(np.float32)).astype(np.float32)
    u = np.zeros((n, 2 * n), np.float32)
    np.add.at(u, (i0, o), 1.0 - lam)
    np.add.at(u, (i1, o), lam)
    return jnp.asarray(u)


def _fused_kernel(w_ref, x_ref, skip_ref, uw_ref, uht_ref, out_ref,
                  y_sc, sum_sc, ssq_sc, scale_sc, shift_sc,
                  *, n_ci, n_co, n_n, d, h, wd, eps):
    p = pl.program_id(0)
    n = pl.program_id(1)
    h2, w2 = 2 * h, 2 * wd
    lanes = n_co * d * w2

    @pl.when((p == 0) & (n == 0))
    def _init():
        sum_sc[...] = jnp.zeros_like(sum_sc)
        ssq_sc[...] = jnp.zeros_like(ssq_sc)

    @pl.when(p == 0)
    def _compute():
        xs = [x_ref[0, ci].reshape(d * h, wd).astype(jnp.bfloat16)
              for ci in range(n_ci)]
        z_list = []
        for c in range(n_co):
            z = xs[0] * w_ref[c, 0].astype(jnp.bfloat16)
            for ci in range(1, n_ci):
                z = z + xs[ci] * w_ref[c, ci].astype(jnp.bfloat16)
            z_list.append(z)
        zcat = jnp.concatenate(z_list, axis=0)

        t = jnp.dot(zcat, uw_ref[...],
                    preferred_element_type=jnp.float32)
        t2 = jnp.concatenate(
            [t[i * h:(i + 1) * h] for i in range(n_co * d)], axis=1)

        y = jnp.dot(uht_ref[...], t2,
                    preferred_element_type=jnp.float32)

        skipcat = jnp.concatenate(
            [skip_ref[0, c, dd] for c in range(n_co) for dd in range(d)],
            axis=1)
        y = y + skipcat

        y_sc[n] = y
        yr = y.reshape(h2 // 8, 8, lanes)
        sum_sc[...] += jnp.sum(yr, axis=0)
        ssq_sc[...] += jnp.sum(yr * yr, axis=0)

    @pl.when((p == 1) & (n == 0))
    def _finalize_stats():
        cnt = jnp.float32(n_n * d * h2 * w2)
        sc_parts, sh_parts = [], []
        for c in range(n_co):
            sl = slice(c * d * w2, (c + 1) * d * w2)
            s = jnp.sum(sum_sc[:, sl])
            q = jnp.sum(ssq_sc[:, sl])
            mean = s / cnt
            var = jnp.maximum(q / cnt - mean * mean, 0.0)
            scl = jax.lax.rsqrt(var + eps)
            sc_parts.append(jnp.full((8, d * w2), scl, jnp.float32))
            sh_parts.append(jnp.full((8, d * w2), -mean * scl, jnp.float32))
        scale_sc[...] = jnp.concatenate(sc_parts, axis=1)
        shift_sc[...] = jnp.concatenate(sh_parts, axis=1)

    @pl.when(p == 1)
    def _apply():
        y = y_sc[n].reshape(h2 // 8, 8, lanes)
        t = (y * scale_sc[...] + shift_sc[...]).reshape(h2, lanes)
        r = jnp.where(t > 0, t, jnp.exp(jnp.minimum(t, 0.0)) - 1.0)
        for c in range(n_co):
            for dd in range(d):
                i = c * d + dd
                out_ref[0, c, dd] = r[:, i * w2:(i + 1) * w2]


def kernel(x, skip, w, b, *, eps=1e-5):
    n_n, n_ci, d, h, wd = x.shape
    n_co = w.shape[0]
    h2, w2 = 2 * h, 2 * wd
    lanes = n_co * d * w2
    del b

    x = x.astype(jnp.float32)
    skip = skip.astype(jnp.float32)
    w32 = w.astype(jnp.float32)

    uw = _upsample_matrix(wd).astype(jnp.bfloat16)
    uht = _upsample_matrix(h).T

    grid = (2, n_n)

    smem_spec = pl.BlockSpec(memory_space=pltpu.MemorySpace.SMEM)
    x_spec = pl.BlockSpec((1, n_ci, d, h, wd),
                          lambda p, n: ((1 - p) * n + p * (n_n - 1),
                                        0, 0, 0, 0))
    skip_spec = pl.BlockSpec((1, n_co, d, h2, w2),
                             lambda p, n: ((1 - p) * n + p * (n_n - 1),
                                           0, 0, 0, 0))
    out_spec = pl.BlockSpec((1, n_co, d, h2, w2),
                            lambda p, n: (p * n, 0, 0, 0, 0))
    uw_spec = pl.BlockSpec((wd, w2), lambda p, n: (0, 0))
    uht_spec = pl.BlockSpec((h2, h), lambda p, n: (0, 0))

    return pl.pallas_call(
        functools.partial(_fused_kernel, n_ci=n_ci, n_co=n_co,
                          n_n=n_n, d=d, h=h, wd=wd, eps=eps),
        out_shape=jax.ShapeDtypeStruct((n_n, n_co, d, h2, w2), jnp.float32),
        grid=grid,
        in_specs=[smem_spec, x_spec, skip_spec, uw_spec, uht_spec],
        out_specs=out_spec,
        scratch_shapes=[
            pltpu.VMEM((n_n, h2, lanes), jnp.float32),
            pltpu.VMEM((8, lanes), jnp.float32),
            pltpu.VMEM((8, lanes), jnp.float32),
            pltpu.VMEM((8, lanes), jnp.float32),
            pltpu.VMEM((8, lanes), jnp.float32),
        ],
        compiler_params=pltpu.CompilerParams(
            dimension_semantics=("arbitrary", "arbitrary")),
    )(w32, x, skip, uw, uht)
